# baseline (speedup 1.0000x reference)
import functools

import jax
import jax.numpy as jnp
from jax import lax
from jax.experimental import pallas as pl
from jax.experimental.pallas import tpu as pltpu
from jax.experimental.pallas import tpu_sc as plsc

BATCH = 4096
SEQ = 200
EMBED_DIM = 128
LANES = 16
NCHUNK = EMBED_DIM // LANES

NUM_CORES = 2
NUM_SUBCORES = 16
NW = NUM_CORES * NUM_SUBCORES
ROWS_PER_W = BATCH // NW
IDX_SPLIT = 2
IDX_CHUNK = SEQ // IDX_SPLIT


NBUF = 3


def _body(x_hbm, table_hbm, out_hbm, idx_v, buf_v, out_v, sem0, sem1, sem2, semi):
    wid = lax.axis_index("s") * NUM_CORES + lax.axis_index("c")
    base = wid * ROWS_PER_W
    sems = (sem0, sem1, sem2)

    pltpu.sync_copy(x_hbm.at[pl.ds(base, 2)], idx_v.at[pl.ds(0, 2)])
    rest = pltpu.async_copy(
        x_hbm.at[pl.ds(base + 2, ROWS_PER_W - 2)],
        idx_v.at[pl.ds(2, ROWS_PER_W - 2)],
        semi,
    )

    inv_n = jnp.float32(1.0 / SEQ)

    def gather_row(r, slot):
        for c in range(IDX_SPLIT):
            pltpu.async_copy(
                table_hbm.at[idx_v.at[r, c]],
                buf_v.at[slot, pl.ds(c * IDX_CHUNK, IDX_CHUNK)],
                sems[slot],
            )

    def wait_row(slot):
        pltpu.make_async_copy(
            table_hbm.at[pl.ds(0, SEQ)], buf_v.at[slot], sems[slot]
        ).wait()

    def accum_row(r, slot):
        def accum(j, carry):
            return tuple(
                carry[c * NCHUNK + d]
                + buf_v[slot, c * IDX_CHUNK + j, pl.ds(d * LANES, LANES)]
                for c in range(IDX_SPLIT)
                for d in range(NCHUNK)
            )

        init = tuple(
            jnp.zeros((LANES,), jnp.float32) for _ in range(IDX_SPLIT * NCHUNK)
        )
        acc = lax.fori_loop(0, IDX_CHUNK, accum, init, unroll=1)
        for d in range(NCHUNK):
            out_v[r, pl.ds(d * LANES, LANES)] = (acc[d] + acc[NCHUNK + d]) * inv_n

    gather_row(0, 0)
    gather_row(1, 1)
    rest.wait()

    main_rows = ROWS_PER_W - (ROWS_PER_W % NBUF)
    half = 66

    def ring_step(r0):
        for b in range(NBUF):
            r = r0 + b
            gather_row(r + 2, (b + 2) % NBUF)
            wait_row(b)
            accum_row(r, b)

    pl.loop(0, half, step=NBUF)(ring_step)
    first_half = pltpu.async_copy(
        out_v.at[pl.ds(0, 64)], out_hbm.at[pl.ds(base, 64)], semi
    )
    pl.loop(half, main_rows, step=NBUF)(ring_step)

    for r in range(main_rows, ROWS_PER_W):
        wait_row(r % NBUF)
        accum_row(r, r % NBUF)

    first_half.wait()
    pltpu.sync_copy(
        out_v.at[pl.ds(64, 64)], out_hbm.at[pl.ds(base + 64, 64)]
    )


@jax.jit
def kernel(x, table):
    x3 = x.astype(jnp.int32).reshape(BATCH, IDX_SPLIT, IDX_CHUNK)
    mesh = plsc.VectorSubcoreMesh(core_axis_name="c", subcore_axis_name="s")
    k = functools.partial(
        pl.kernel,
        out_type=jax.ShapeDtypeStruct((BATCH, EMBED_DIM), jnp.float32),
        mesh=mesh,
        scratch_types=[
            pltpu.VMEM((ROWS_PER_W, IDX_SPLIT, IDX_CHUNK), jnp.int32),
            pltpu.VMEM((NBUF, SEQ, EMBED_DIM), jnp.float32),
            pltpu.VMEM((ROWS_PER_W, EMBED_DIM), jnp.float32),
            pltpu.SemaphoreType.DMA,
            pltpu.SemaphoreType.DMA,
            pltpu.SemaphoreType.DMA,
            pltpu.SemaphoreType.DMA,
        ],
    )(_body)
    return k(x3, table)


# speedup vs baseline: 1.0051x; 1.0051x over previous
import functools

import jax
import jax.numpy as jnp
from jax import lax
from jax.experimental import pallas as pl
from jax.experimental.pallas import tpu as pltpu
from jax.experimental.pallas import tpu_sc as plsc

BATCH = 4096
SEQ = 200
EMBED_DIM = 128
LANES = 16
NCHUNK = EMBED_DIM // LANES

NUM_CORES = 2
NUM_SUBCORES = 16
NW = NUM_CORES * NUM_SUBCORES
ROWS_PER_W = BATCH // NW
IDX_SPLIT = 2
IDX_CHUNK = SEQ // IDX_SPLIT


NBUF = 3


def _body(x_hbm, table_hbm, out_hbm, idx_v, buf_v, out_v, sem0, sem1, sem2, semi):
    wid = lax.axis_index("s") * NUM_CORES + lax.axis_index("c")
    base = wid * ROWS_PER_W
    sems = (sem0, sem1, sem2)

    pltpu.sync_copy(x_hbm.at[pl.ds(base, 2)], idx_v.at[pl.ds(0, 2)])
    rest = pltpu.async_copy(
        x_hbm.at[pl.ds(base + 2, ROWS_PER_W - 2)],
        idx_v.at[pl.ds(2, ROWS_PER_W - 2)],
        semi,
    )

    inv_n = jnp.float32(1.0 / SEQ)

    def gather_row(r, slot):
        for c in range(IDX_SPLIT):
            pltpu.async_copy(
                table_hbm.at[idx_v.at[r, c]],
                buf_v.at[slot, pl.ds(c * IDX_CHUNK, IDX_CHUNK)],
                sems[slot],
            )

    def wait_row(slot):
        pltpu.make_async_copy(
            table_hbm.at[pl.ds(0, SEQ)], buf_v.at[slot], sems[slot]
        ).wait()

    def accum_row(r, slot):
        def accum(j, carry):
            return tuple(
                carry[c * NCHUNK + d]
                + buf_v[slot, c * IDX_CHUNK + j, pl.ds(d * LANES, LANES)]
                for c in range(IDX_SPLIT)
                for d in range(NCHUNK)
            )

        init = tuple(
            jnp.zeros((LANES,), jnp.float32) for _ in range(IDX_SPLIT * NCHUNK)
        )
        acc = lax.fori_loop(0, IDX_CHUNK, accum, init, unroll=1)
        for d in range(NCHUNK):
            out_v[r, pl.ds(d * LANES, LANES)] = (acc[d] + acc[NCHUNK + d]) * inv_n

    gather_row(0, 0)
    gather_row(1, 1)
    rest.wait()

    main_rows = ROWS_PER_W - (ROWS_PER_W % NBUF)

    @pl.loop(0, main_rows, step=NBUF)
    def ring(r0):
        for b in range(NBUF):
            r = r0 + b
            gather_row(r + 2, (b + 2) % NBUF)
            wait_row(b)
            accum_row(r, b)

    for r in range(main_rows, ROWS_PER_W):
        wait_row(r % NBUF)
        accum_row(r, r % NBUF)

    pltpu.sync_copy(out_v, out_hbm.at[pl.ds(base, ROWS_PER_W)])


@jax.jit
def kernel(x, table):
    x3 = x.astype(jnp.int32).reshape(BATCH, IDX_SPLIT, IDX_CHUNK)
    mesh = plsc.VectorSubcoreMesh(core_axis_name="c", subcore_axis_name="s")
    k = functools.partial(
        pl.kernel,
        out_type=jax.ShapeDtypeStruct((BATCH, EMBED_DIM), jnp.float32),
        mesh=mesh,
        scratch_types=[
            pltpu.VMEM((ROWS_PER_W, IDX_SPLIT, IDX_CHUNK), jnp.int32),
            pltpu.VMEM((NBUF, SEQ, EMBED_DIM), jnp.float32),
            pltpu.VMEM((ROWS_PER_W, EMBED_DIM), jnp.float32),
            pltpu.SemaphoreType.DMA,
            pltpu.SemaphoreType.DMA,
            pltpu.SemaphoreType.DMA,
            pltpu.SemaphoreType.DMA,
        ],
    )(_body)
    return k(x3, table)
